# baseline (speedup 1.0000x reference)
import jax
import jax.numpy as jnp
from jax import lax
from jax.experimental import pallas as pl
from jax.experimental.pallas import tpu as pltpu
from jax.experimental.pallas import tpu_sc as plsc

_T_END = 4096
_M = 2048
_NCHUNK = 4
_C = _M // _NCHUNK


def _row_lookup_body(u_hbm, t_hbm, z_hbm, out_hbm, t_s, row_sp, in_sems, out_sems):
    pltpu.sync_copy(t_hbm, t_s)
    t = t_s[0]
    safe_t = jnp.minimum(t, _T_END - 1)
    valid = t < _T_END

    @pl.when(valid)
    def _copy_row():
        loads = []
        for i in range(_NCHUNK):
            sl = pl.ds(i * _C, _C)
            loads.append(
                pltpu.async_copy(u_hbm.at[safe_t, sl], row_sp.at[sl], in_sems.at[i])
            )
        stores = []
        for i in range(_NCHUNK):
            sl = pl.ds(i * _C, _C)
            loads[i].wait()
            stores.append(
                pltpu.async_copy(row_sp.at[sl], out_hbm.at[sl], out_sems.at[i])
            )
        for s in stores:
            s.wait()

    @pl.when(jnp.logical_not(valid))
    def _copy_zeros():
        pltpu.sync_copy(z_hbm, row_sp)
        pltpu.sync_copy(row_sp, out_hbm)


def kernel(u, t):
    t_vec = jnp.reshape(jnp.asarray(t, jnp.int32), (1,))
    zeros_row = jnp.zeros((_M,), jnp.float32)
    f = pl.kernel(
        _row_lookup_body,
        out_type=jax.ShapeDtypeStruct((_M,), jnp.float32),
        mesh=plsc.ScalarSubcoreMesh(axis_name="c", num_cores=1),
        scratch_types=[
            pltpu.SMEM((1,), jnp.int32),
            pltpu.VMEM_SHARED((_M,), jnp.float32),
            pltpu.SemaphoreType.DMA((_NCHUNK,)),
            pltpu.SemaphoreType.DMA((_NCHUNK,)),
        ],
    )
    return f(u, t_vec, zeros_row)
